# baseline (speedup 1.0000x reference)
import functools

import jax
import jax.numpy as jnp
from jax import lax
from jax.experimental import pallas as pl
from jax.experimental.pallas import tpu as pltpu
from jax.experimental.pallas import tpu_sc as plsc

_LANES = 16
_NUM_CORES = 2
_NUM_SUBCORES = 16
_NUM_WORKERS = _NUM_CORES * _NUM_SUBCORES


def _make_sc_kernel(n: int, h: int, s: int):
    rows_per_w = n // _NUM_WORKERS
    groups = h // _LANES

    mesh = plsc.VectorSubcoreMesh(core_axis_name="c", subcore_axis_name="s")

    @functools.partial(
        pl.kernel,
        mesh=mesh,
        compiler_params=pltpu.CompilerParams(
            needs_layout_passes=False, use_tc_tiling_on_sc=False
        ),
        out_type=(
            jax.ShapeDtypeStruct((n, h), jnp.float32),
            jax.ShapeDtypeStruct((n, h), jnp.int32),
        ),
        scratch_types=[
            pltpu.VMEM((h, s), jnp.float32),
            pltpu.VMEM((h,), jnp.float32),
            pltpu.VMEM((h,), jnp.float32),
            pltpu.VMEM((h,), jnp.int32),
        ],
    )
    def sc_kernel(d_hbm, y_hbm, val_hbm, idx_hbm, dbuf, ybuf, vbuf, ibuf):
        wid = lax.axis_index("s") * _NUM_CORES + lax.axis_index("c")
        row0 = wid * rows_per_w
        lane = lax.iota(jnp.int32, _LANES)
        gidx0 = lane * s

        def row_body(r, _):
            pltpu.sync_copy(d_hbm.at[r], dbuf)
            pltpu.sync_copy(y_hbm.at[r], ybuf)

            def group_body(g, _):
                jvec = lane + g * _LANES
                zero = jnp.zeros((_LANES,), jnp.int32)
                v = plsc.load_gather(dbuf, [jvec, zero])
                best_v = v
                best_a = jnp.abs(v)
                best_i = jnp.zeros((_LANES,), jnp.int32)
                for k in range(1, s):
                    v = plsc.load_gather(dbuf, [jvec, zero + k])
                    a = jnp.abs(v)
                    p = a > best_a
                    best_a = jnp.where(p, a, best_a)
                    best_v = jnp.where(p, v, best_v)
                    best_i = jnp.where(p, k, best_i)
                gate = 1.0 / (1.0 + jnp.exp(-best_v))
                yv = ybuf[pl.ds(g * _LANES, _LANES)]
                vbuf[pl.ds(g * _LANES, _LANES)] = yv * gate
                ibuf[pl.ds(g * _LANES, _LANES)] = best_i
                return 0

            lax.fori_loop(0, groups, group_body, 0)
            pltpu.sync_copy(vbuf, val_hbm.at[r])
            pltpu.sync_copy(ibuf, idx_hbm.at[r])
            return 0

        lax.fori_loop(row0, row0 + rows_per_w, row_body, 0)

    return sc_kernel


def kernel(y, dendrite_activations):
    n, h = y.shape
    s = dendrite_activations.shape[2]
    sc = _make_sc_kernel(n, h, s)
    return sc(dendrite_activations, y)
